# Initial kernel scaffold; baseline (speedup 1.0000x reference)
import jax
import jax.numpy as jnp
from jax.experimental import pallas as pl


def kernel(x_og, x_cg, pos_og, pos_cg, og_to_cg_edge_index, og_to_cg_edge_attr, edge_index_cg, x_og_batch, x_cg_batch, params):
    raise NotImplementedError("write your pallas kernel here")


# speedup vs baseline: 1.0013x; 1.0013x over previous
import functools

import jax
import jax.numpy as jnp
from jax import lax
from jax.experimental import pallas as pl
from jax.experimental.pallas import tpu as pltpu
from jax.experimental.pallas import tpu_sc as plsc

H = 128
POS = 2
EPS = 1e-5
NC = 2
NS = 16
NW = NC * NS
CH = 128

F32 = jnp.float32


def _rup(n, m):
    return ((n + m - 1) // m) * m


def _sc_gather2(tab_a, tab_b, idx_a, idx_b):
    ep = idx_a.shape[0]
    wa = tab_a.shape[1]
    wb = tab_b.shape[1]
    per_tile = ep // NW
    nch = per_tile // CH
    mesh = plsc.VectorSubcoreMesh(core_axis_name="c", subcore_axis_name="s")

    @functools.partial(
        pl.kernel, mesh=mesh,
        out_type=(jax.ShapeDtypeStruct((ep, wa), F32),
                  jax.ShapeDtypeStruct((ep, wb), F32)),
        scratch_types=[
            pltpu.VMEM((CH,), jnp.int32),
            pltpu.VMEM((CH,), jnp.int32),
            pltpu.VMEM((CH, wa), F32),
            pltpu.VMEM((CH, wb), F32),
            pltpu.SemaphoreType.DMA,
            pltpu.SemaphoreType.DMA,
        ],
    )
    def k(ta_hbm, tb_hbm, ia_hbm, ib_hbm, oa_hbm, ob_hbm,
          ia_v, ib_v, ra_v, rb_v, sem_a, sem_b):
        wid = lax.axis_index("s") * NC + lax.axis_index("c")
        base = wid * per_tile

        def body(ch, carry):
            off = base + ch * CH
            pltpu.sync_copy(ia_hbm.at[pl.ds(off, CH)], ia_v)
            pltpu.sync_copy(ib_hbm.at[pl.ds(off, CH)], ib_v)
            cp_a = pltpu.async_copy(ta_hbm.at[ia_v], ra_v, sem_a)
            cp_b = pltpu.async_copy(tb_hbm.at[ib_v], rb_v, sem_b)
            cp_a.wait()
            cp_b.wait()
            pltpu.sync_copy(ra_v, oa_hbm.at[pl.ds(off, CH)])
            pltpu.sync_copy(rb_v, ob_hbm.at[pl.ds(off, CH)])
            return carry

        lax.fori_loop(0, nch, body, 0)

    return k(tab_a, tab_b, idx_a, idx_b)


def _sc_scatter_add(data, idx, zeros, n_pad, w):
    ep = idx.shape[0]
    per_tile = ep // NW
    nch = per_tile // CH
    mesh = plsc.VectorSubcoreMesh(core_axis_name="c", subcore_axis_name="s")

    @functools.partial(
        pl.kernel, mesh=mesh,
        out_type=jax.ShapeDtypeStruct((NC, n_pad, w), F32),
        scratch_types=[
            pltpu.VMEM((CH,), jnp.int32),
            pltpu.VMEM((CH, w), F32),
            pltpu.VMEM_SHARED((n_pad, w), F32),
        ],
    )
    def k(d_hbm, i_hbm, z_hbm, o_hbm, i_v, r_v, acc_sh):
        c = lax.axis_index("c")
        s = lax.axis_index("s")
        wid = s * NC + c

        @pl.when(s == 0)
        def _():
            pltpu.sync_copy(z_hbm, acc_sh)

        plsc.subcore_barrier()
        base = wid * per_tile

        def body(ch, carry):
            off = base + ch * CH
            pltpu.sync_copy(i_hbm.at[pl.ds(off, CH)], i_v)
            pltpu.sync_copy(d_hbm.at[pl.ds(off, CH)], r_v)
            pltpu.sync_copy(r_v, acc_sh.at[i_v], add=True)
            return carry

        lax.fori_loop(0, nch, body, 0)
        plsc.subcore_barrier()

        @pl.when(s == 0)
        def _():
            pltpu.sync_copy(acc_sh, o_hbm.at[c])

    return k(data, idx, zeros)


def _sc_gather_scatter(table, idx_s, idx_d, zeros, n_pad):
    ep = idx_s.shape[0]
    per_tile = ep // NW
    nch = per_tile // CH
    mesh = plsc.VectorSubcoreMesh(core_axis_name="c", subcore_axis_name="s")

    @functools.partial(
        pl.kernel, mesh=mesh,
        out_type=jax.ShapeDtypeStruct((NC, n_pad, H), F32),
        scratch_types=[
            pltpu.VMEM((CH,), jnp.int32),
            pltpu.VMEM((CH,), jnp.int32),
            pltpu.VMEM((CH, H), F32),
            pltpu.VMEM_SHARED((n_pad, H), F32),
            pltpu.SemaphoreType.DMA,
        ],
    )
    def k(t_hbm, is_hbm, id_hbm, z_hbm, o_hbm, is_v, id_v, r_v, acc_sh, sem):
        c = lax.axis_index("c")
        s = lax.axis_index("s")
        wid = s * NC + c

        @pl.when(s == 0)
        def _():
            pltpu.sync_copy(z_hbm, acc_sh)

        plsc.subcore_barrier()
        base = wid * per_tile

        def body(ch, carry):
            off = base + ch * CH
            pltpu.sync_copy(is_hbm.at[pl.ds(off, CH)], is_v)
            pltpu.sync_copy(id_hbm.at[pl.ds(off, CH)], id_v)
            pltpu.async_copy(t_hbm.at[is_v], r_v, sem).wait()
            pltpu.sync_copy(r_v, acc_sh.at[id_v], add=True)
            return carry

        lax.fori_loop(0, nch, body, 0)
        plsc.subcore_barrier()

        @pl.when(s == 0)
        def _():
            pltpu.sync_copy(acc_sh, o_hbm.at[c])

    return k(table, idx_s, idx_d, zeros)


def _sc_counts(idx, ones_blk, zeros, n_pad):
    ep = idx.shape[0]
    per_tile = ep // NW
    nch = per_tile // CH
    mesh = plsc.VectorSubcoreMesh(core_axis_name="c", subcore_axis_name="s")

    @functools.partial(
        pl.kernel, mesh=mesh,
        out_type=jax.ShapeDtypeStruct((NC, n_pad, H), F32),
        scratch_types=[
            pltpu.VMEM((CH,), jnp.int32),
            pltpu.VMEM((CH, H), F32),
            pltpu.VMEM_SHARED((n_pad, H), F32),
        ],
    )
    def k(i_hbm, one_hbm, z_hbm, o_hbm, i_v, one_v, acc_sh):
        c = lax.axis_index("c")
        s = lax.axis_index("s")
        wid = s * NC + c
        pltpu.sync_copy(one_hbm, one_v)

        @pl.when(s == 0)
        def _():
            pltpu.sync_copy(z_hbm, acc_sh)

        plsc.subcore_barrier()
        base = wid * per_tile

        def body(ch, carry):
            off = base + ch * CH
            pltpu.sync_copy(i_hbm.at[pl.ds(off, CH)], i_v)
            pltpu.sync_copy(one_v, acc_sh.at[i_v], add=True)
            return carry

        lax.fori_loop(0, nch, body, 0)
        plsc.subcore_barrier()

        @pl.when(s == 0)
        def _():
            pltpu.sync_copy(acc_sh, o_hbm.at[c])

    return k(idx, ones_blk, zeros)


def _bn_masked(x, mask, n):
    xm = jnp.where(mask, x, 0.0)
    m = jnp.sum(xm, axis=0, keepdims=True) / n
    d = jnp.where(mask, x - m, 0.0)
    v = jnp.sum(d * d, axis=0, keepdims=True) / n
    return (x - m) / jnp.sqrt(v + EPS)


def _mask(n_pad, n):
    return lax.broadcasted_iota(jnp.int32, (n_pad, 1), 0) < n


def _tc_call(body, out_shape, *args):
    return pl.pallas_call(body, out_shape=out_shape)(*args)


def _tc_mlp(x_p, ws, bs, n, plain_last):
    n_pad = x_p.shape[0]
    nl = len(ws)

    def body(*refs):
        x_ref = refs[0]
        o_ref = refs[-1]
        mask = _mask(n_pad, n)
        h = x_ref[...]
        for i in range(nl):
            w = refs[1 + 2 * i][...]
            b = refs[2 + 2 * i][...]
            h = jnp.dot(h, w) + b
            if (i < nl - 1) or (not plain_last):
                h = jax.nn.relu(_bn_masked(h, mask, n))
        o_ref[...] = jnp.where(mask, h, 0.0)

    args = [x_p]
    for w, b in zip(ws, bs):
        args += [w, b.reshape(1, H)]
    return _tc_call(body, jax.ShapeDtypeStruct((n_pad, H), F32), *args)


def _tc_tables(x_dst_p, x_src_p, wx):
    nd = x_dst_p.shape[0]
    ns = x_src_p.shape[0]

    def body(xd, xs, wx_ref, ti_ref, tj_ref):
        w = wx_ref[...]
        ti_ref[...] = jnp.dot(xd[...], w[0:H, :])
        tj_ref[...] = jnp.dot(xs[...], w[H:2 * H, :])

    out_shape = (jax.ShapeDtypeStruct((nd, H), F32),
                 jax.ShapeDtypeStruct((ns, H), F32))
    return pl.pallas_call(body, out_shape=out_shape)(x_dst_p, x_src_p, wx)


def _tc_pd(gpd, gps):
    ep = gpd.shape[0]
    be_ = 2048
    nb = ep // be_

    def body(a_ref, b_ref, o_ref):
        o_ref[...] = a_ref[...][:, 0:16] - b_ref[...][:, 0:16]

    blk = lambda j: (j, 0)
    return pl.pallas_call(
        body, grid=(nb,),
        in_specs=[pl.BlockSpec((be_, H), blk), pl.BlockSpec((be_, H), blk)],
        out_specs=pl.BlockSpec((be_, 16), blk),
        out_shape=jax.ShapeDtypeStruct((ep, 16), F32),
    )(gpd, gps)


def _tc_edge(g1, g2, pd16, sign, eattr_p, p, n_edges):
    ep = g1.shape[0]
    be_ = 2048
    nb = ep // be_
    n_pr = ep - n_edges

    def body(g1_ref, g2_ref, pd_ref, ea_ref,
             wp_ref, we_ref, wc1_ref, wc2_ref,
             bx_ref, bp_ref, be_ref, bc1_ref, bc2_ref,
             h2_ref, st_ref):
        e = pl.program_id(0)
        b = pl.program_id(1)
        wc1 = wc1_ref[...]
        xj = g1_ref[...] + g2_ref[...] + bx_ref[...]
        pd = pd_ref[...][:, 0:POS] * sign
        pe = jnp.dot(pd, wp_ref[...]) + bp_ref[...]
        ee = jnp.dot(ea_ref[...], we_ref[...]) + be_ref[...]
        lin1 = (jnp.dot(xj, wc1[0:H, :]) + jnp.dot(pe, wc1[H:2 * H, :])
                + jnp.dot(ee, wc1[2 * H:3 * H, :]) + bc1_ref[...])

        def pad_lin1():
            return (jnp.dot(bx_ref[...], wc1[0:H, :])
                    + jnp.dot(bp_ref[...], wc1[H:2 * H, :])
                    + jnp.dot(be_ref[...], wc1[2 * H:3 * H, :])
                    + bc1_ref[...])

        @pl.when((e == 0) & (b == 0))
        def _():
            st_ref[...] = jnp.zeros((8, H), F32)

        @pl.when(e == 0)
        def _():
            st_ref[0:1, :] += jnp.sum(lin1, axis=0, keepdims=True)
            h2_ref[...] = lin1

            @pl.when(b == nb - 1)
            def _():
                l1p = pad_lin1()
                st_ref[4:5, :] = (st_ref[0:1, :] - n_pr * l1p) / n_edges

        @pl.when(e == 1)
        def _():
            d = lin1 - st_ref[4:5, :]
            st_ref[1:2, :] += jnp.sum(d * d, axis=0, keepdims=True)
            h2_ref[...] = lin1

            @pl.when(b == nb - 1)
            def _():
                dp = pad_lin1() - st_ref[4:5, :]
                q1 = st_ref[1:2, :] - n_pr * (dp * dp)
                st_ref[5:6, :] = jnp.sqrt(q1 / n_edges + EPS)

        @pl.when(e == 2)
        def _():
            t = jax.nn.relu((lin1 - st_ref[4:5, :]) / st_ref[5:6, :])
            lin2 = jnp.dot(t, wc2_ref[...]) + bc2_ref[...]
            st_ref[2:3, :] += jnp.sum(lin2, axis=0, keepdims=True)
            h2_ref[...] = lin2

            @pl.when(b == nb - 1)
            def _():
                tp = jax.nn.relu((pad_lin1() - st_ref[4:5, :])
                                 / st_ref[5:6, :])
                l2p = jnp.dot(tp, wc2_ref[...]) + bc2_ref[...]
                st_ref[6:7, :] = (st_ref[2:3, :] - n_pr * l2p) / n_edges

        @pl.when(e == 3)
        def _():
            t = jax.nn.relu((lin1 - st_ref[4:5, :]) / st_ref[5:6, :])
            lin2 = jnp.dot(t, wc2_ref[...]) + bc2_ref[...]
            d2v = lin2 - st_ref[6:7, :]
            st_ref[3:4, :] += jnp.sum(d2v * d2v, axis=0, keepdims=True)
            h2_ref[...] = jax.nn.relu(d2v)

            @pl.when(b == nb - 1)
            def _():
                tp = jax.nn.relu((pad_lin1() - st_ref[4:5, :])
                                 / st_ref[5:6, :])
                l2p = jnp.dot(tp, wc2_ref[...]) + bc2_ref[...]
                dp = l2p - st_ref[6:7, :]
                q2 = st_ref[3:4, :] - n_pr * (dp * dp)
                st_ref[7:8, :] = jnp.sqrt(q2 / n_edges + EPS)

    grid = (4, nb)
    blk = lambda i, j: (j, 0)
    cst = lambda i, j: (0, 0)
    out_shape = (jax.ShapeDtypeStruct((ep, H), F32),
                 jax.ShapeDtypeStruct((8, H), F32))
    return pl.pallas_call(
        body,
        grid=grid,
        in_specs=[pl.BlockSpec((be_, H), blk),
                  pl.BlockSpec((be_, H), blk),
                  pl.BlockSpec((be_, 16), blk),
                  pl.BlockSpec((be_, POS), blk),
                  pl.BlockSpec((POS, H), cst),
                  pl.BlockSpec((POS, H), cst),
                  pl.BlockSpec((3 * H, H), cst),
                  pl.BlockSpec((H, H), cst),
                  pl.BlockSpec((1, H), cst),
                  pl.BlockSpec((1, H), cst),
                  pl.BlockSpec((1, H), cst),
                  pl.BlockSpec((1, H), cst),
                  pl.BlockSpec((1, H), cst)],
        out_specs=(pl.BlockSpec((be_, H), blk),
                   pl.BlockSpec((8, H), cst)),
        out_shape=out_shape,
    )(g1, g2, pd16, eattr_p,
      p['Wp'], p['We'], p['Wc1'], p['Wc2'],
      p['bx'].reshape(1, H), p['bp'].reshape(1, H), p['be'].reshape(1, H),
      p['bc1'].reshape(1, H), p['bc2'].reshape(1, H))


def _tc_cnt_fin(c1, c2, c3):
    def body(a_ref, b_ref, c_ref, oa, ob, oc):
        for r, o in ((a_ref, oa), (b_ref, ob), (c_ref, oc)):
            v = r[...]
            o[...] = v[0, :, 0:1] + v[1, :, 0:1]

    out_shape = tuple(jax.ShapeDtypeStruct((c.shape[1], 1), F32)
                      for c in (c1, c2, c3))
    return pl.pallas_call(body, out_shape=out_shape)(c1, c2, c3)


def _tc_seg_fin(acc, cnt, stats, n):
    n_pad = acc.shape[1]

    def body(a_ref, c_ref, st_ref, o_ref):
        mask = _mask(n_pad, n)
        a = a_ref[...]
        s = a[0] + a[1]
        cnt_v = jnp.maximum(c_ref[...], 1.0)
        o_ref[...] = jnp.where(mask, (s / cnt_v) / st_ref[7:8, :], 0.0)

    return _tc_call(body, jax.ShapeDtypeStruct((n_pad, H), F32),
                    acc, cnt, stats)


def _tc_conv_spread(acc2, cnt2, conv_w, conv_b, prev_cg_p, h_og_p, wx_s, n_cg):
    n_cgp = acc2.shape[1]
    n_ogp = h_og_p.shape[0]

    def body(a_ref, c_ref, cw, cb, pcg, hog, wx_ref,
             hcg_ref, ti_ref, tj_ref):
        mask = _mask(n_cgp, n_cg)
        a = a_ref[...]
        agg = (a[0] + a[1]) / jnp.maximum(c_ref[...], 1.0)
        h_cg_new = jnp.where(mask, jax.nn.relu(jnp.dot(agg, cw[...])
                                               + cb[...]), 0.0)
        hcg_ref[...] = h_cg_new
        x_src = h_cg_new + pcg[...]
        w = wx_ref[...]
        ti_ref[...] = jnp.dot(hog[...], w[0:H, :])
        tj_ref[...] = jnp.dot(x_src, w[H:2 * H, :])

    out_shape = (jax.ShapeDtypeStruct((n_cgp, H), F32),
                 jax.ShapeDtypeStruct((n_ogp, H), F32),
                 jax.ShapeDtypeStruct((n_cgp, H), F32))
    return pl.pallas_call(body, out_shape=out_shape)(
        acc2, cnt2, conv_w, conv_b.reshape(1, H), prev_cg_p, h_og_p, wx_s)


def _tc_og_fin(acc_s, cnt_s, stats, h_og_p, prev_og_p, gate_w, gate_b,
               lin_ws, lin_bs, n_og):
    n_ogp = acc_s.shape[1]

    def body(a_ref, c_ref, st_ref, h_ref, pv_ref, gw, gb,
             w1, b1, w2, b2, o_ref):
        mask = _mask(n_ogp, n_og)
        a = a_ref[...]
        s = a[0] + a[1]
        spread = (s / jnp.maximum(c_ref[...], 1.0)) / st_ref[7:8, :]
        spread = jnp.where(mask, spread, 0.0)
        h = h_ref[...]
        gwv = gw[...]
        gate = jax.nn.sigmoid(jnp.dot(h, gwv[0:H, :])
                              + jnp.dot(spread, gwv[H:2 * H, :]) + gb[...])
        h = gate * h + (1.0 - gate) * spread
        h = jax.nn.gelu(_bn_masked(h, mask, n_og))
        h = jax.nn.relu(_bn_masked(jnp.dot(h, w1[...]) + b1[...], mask, n_og))
        h = jax.nn.relu(_bn_masked(jnp.dot(h, w2[...]) + b2[...], mask, n_og))
        o_ref[...] = jnp.where(mask, h + pv_ref[...], 0.0)

    return _tc_call(body, jax.ShapeDtypeStruct((n_ogp, H), F32),
                    acc_s, cnt_s, stats, h_og_p, prev_og_p,
                    gate_w, gate_b.reshape(1, H),
                    lin_ws[0], lin_bs[0].reshape(1, H),
                    lin_ws[1], lin_bs[1].reshape(1, H))


def kernel(x_og, x_cg, pos_og, pos_cg, og_to_cg_edge_index, og_to_cg_edge_attr,
           edge_index_cg, x_og_batch, x_cg_batch, params):
    n_og, _ = x_og.shape
    n_cg, _ = x_cg.shape
    e1 = og_to_cg_edge_index.shape[1]
    e2 = edge_index_cg.shape[1]
    layers = params['layers']

    n_ogp = _rup(n_og + 1, 16)
    n_cgp = _rup(n_cg + 1, 16)
    e1p = _rup(e1, NW * CH)
    e2p = _rup(e2, NW * CH)

    src1 = og_to_cg_edge_index[0]
    dst1 = og_to_cg_edge_index[1]
    s2 = edge_index_cg[0]
    d2 = edge_index_cg[1]
    src1p = jnp.pad(src1, (0, e1p - e1), constant_values=n_og).astype(jnp.int32)
    dst1p = jnp.pad(dst1, (0, e1p - e1), constant_values=n_cg).astype(jnp.int32)
    s2p = jnp.pad(s2, (0, e2p - e2), constant_values=n_cg).astype(jnp.int32)
    d2p = jnp.pad(d2, (0, e2p - e2), constant_values=n_cg).astype(jnp.int32)
    eattr_p = jnp.pad(og_to_cg_edge_attr, ((0, e1p - e1), (0, 0)))

    x_og_p = jnp.pad(x_og, ((0, n_ogp - n_og), (0, 0)))
    x_cg_p = jnp.pad(x_cg, ((0, n_cgp - n_cg), (0, 0)))
    pos128_og = jnp.pad(pos_og, ((0, n_ogp - n_og), (0, H - POS)))
    pos128_cg = jnp.pad(pos_cg, ((0, n_cgp - n_cg), (0, H - POS)))

    zeros_og = jnp.zeros((n_ogp, H), F32)
    zeros_cg = jnp.zeros((n_cgp, H), F32)
    ones_blk = jnp.ones((CH, H), F32)

    cnt_dst_raw = _sc_counts(dst1p, ones_blk, zeros_cg, n_cgp)
    cnt_src_raw = _sc_counts(src1p, ones_blk, zeros_og, n_ogp)
    cnt_d2_raw = _sc_counts(d2p, ones_blk, zeros_cg, n_cgp)
    cnt_dst, cnt_src, cnt_d2 = _tc_cnt_fin(cnt_dst_raw, cnt_src_raw,
                                           cnt_d2_raw)

    gp_cg, gp_og = _sc_gather2(pos128_cg, pos128_og, dst1p, src1p)
    pd16 = _tc_pd(gp_cg, gp_og)

    h_og_p = _tc_mlp(x_og_p, params['og_proj']['Ws'], params['og_proj']['bs'],
                     n_og, plain_last=False)
    h_cg_p = _tc_mlp(x_cg_p, params['cg_proj']['Ws'], params['cg_proj']['bs'],
                     n_cg, plain_last=False)

    for lp in layers:
        prev_og_p = h_og_p
        prev_cg_p = h_cg_p

        ti_cg, tj_og = _tc_tables(h_cg_p, h_og_p, lp['coars']['Wx'])
        g1, g2 = _sc_gather2(ti_cg, tj_og, dst1p, src1p)
        h2_c, st_c = _tc_edge(g1, g2, pd16, 1.0, eattr_p, lp['coars'], e1)
        acc_c = _sc_scatter_add(h2_c, dst1p, zeros_cg, n_cgp, H)
        hmsg_p = _tc_seg_fin(acc_c, cnt_dst, st_c, n_cg)

        acc2 = _sc_gather_scatter(hmsg_p, s2p, d2p, zeros_cg, n_cgp)
        h_cg_p, ti_og, tj_cg = _tc_conv_spread(
            acc2, cnt_d2, lp['conv_W'], lp['conv_b'], prev_cg_p,
            h_og_p, lp['spread']['Wx'], n_cg)

        g1s, g2s = _sc_gather2(ti_og, tj_cg, src1p, dst1p)
        h2_s, st_s = _tc_edge(g1s, g2s, pd16, -1.0, eattr_p, lp['spread'],
                              e1)
        acc_s = _sc_scatter_add(h2_s, src1p, zeros_og, n_ogp, H)
        h_og_p = _tc_og_fin(acc_s, cnt_src, st_s, h_og_p, prev_og_p,
                            params['gate_W'], params['gate_b'],
                            lp['og_lin']['Ws'], lp['og_lin']['bs'], n_og)

    out_p = _tc_mlp(h_og_p, params['out']['Ws'], params['out']['bs'],
                    n_og, plain_last=True)
    return out_p[:n_og]
